# Initial kernel scaffold; baseline (speedup 1.0000x reference)
import jax
import jax.numpy as jnp
from jax.experimental import pallas as pl


def kernel(x, Wl, bl, Wr, br, att, bias):
    raise NotImplementedError("write your pallas kernel here")


# speedup vs baseline: 3.7373x; 3.7373x over previous
import functools
import math

import jax
import jax.numpy as jnp
from jax import lax
from jax.experimental import pallas as pl
from jax.experimental.pallas import tpu as pltpu
from jax.experimental.pallas import tpu_sc as plsc

BN = 10000
DN = 128
KN = 16
BP = 10240
RB = 256
NBLK = BP // RB
EP = KN * BP
NEG = 0.2
NEG_INF = float("-inf")


def _k1_body(xp_ref, wlt_ref, bl_ref, wrt_ref, br_ref, att_ref,
             xl_ref, xr_ref, gs_ref, self2_ref):
    xp = xp_ref[...]
    xl = jnp.dot(xp, wlt_ref[...], preferred_element_type=jnp.float32) + bl_ref[...]
    xr = jnp.dot(xp, wrt_ref[...], preferred_element_type=jnp.float32) + br_ref[...]
    attc = att_ref[...].reshape(DN, 1)
    aabs = jnp.abs(attc)
    al = jnp.dot(xl, attc, preferred_element_type=jnp.float32)
    p = jnp.dot(jnp.abs(xl), aabs, preferred_element_type=jnp.float32)
    ar = jnp.dot(xr, attc, preferred_element_type=jnp.float32)
    q = jnp.dot(jnp.abs(xr), aabs, preferred_element_type=jnp.float32)
    rowid = lax.broadcasted_iota(jnp.int32, (BP, 1), 0)
    ok = rowid < BN
    gsrc = jnp.max(jnp.where(ok, 0.6 * al + 0.4 * p, jnp.float32(-1e30)))
    gdst = jnp.max(jnp.where(ok, 0.6 * ar + 0.4 * q, jnp.float32(-1e30)))
    gshift = gsrc + gdst
    z = xl + xr
    lr = jnp.where(z > 0, z, NEG * z)
    ls = jnp.dot(lr, attc, preferred_element_type=jnp.float32)
    selfw = jnp.exp(ls - gshift)
    xl_ref[...] = xl
    xr_ref[...] = xr
    gs_ref[...] = jnp.full((8, DN), gshift, jnp.float32)
    self2_ref[...] = jnp.concatenate(
        [selfw * xl, jnp.broadcast_to(selfw, (BP, DN))], axis=1)


def _k1(xp, wlt, bl2, wrt, br2, att2):
    return pl.pallas_call(
        _k1_body,
        out_shape=(
            jax.ShapeDtypeStruct((BP, DN), jnp.float32),
            jax.ShapeDtypeStruct((BP, DN), jnp.float32),
            jax.ShapeDtypeStruct((8, DN), jnp.float32),
            jax.ShapeDtypeStruct((BP, 2 * DN), jnp.float32),
        ),
    )(xp, wlt, bl2, wrt, br2, att2)


def _k2_body(xrow_ref, xt_ref, idx_ref, s_ref):
    s_ref[...] = jnp.dot(xrow_ref[...], xt_ref[...],
                         preferred_element_type=jnp.float32)
    colid = lax.broadcasted_iota(jnp.int32, (RB, BP), 1)
    s = jnp.where(colid < BN, s_ref[...], NEG_INF)
    big = jnp.int32(1 << 30)
    for k in range(KN):
        m = jnp.max(s, axis=1, keepdims=True)
        a = jnp.min(jnp.where(s == m, colid, big), axis=1, keepdims=True)
        idx_ref[:, k:k + 1] = a
        s = jnp.where(colid == a, NEG_INF, s)


def _k2(xp, xt):
    return pl.pallas_call(
        _k2_body,
        grid=(NBLK,),
        in_specs=[
            pl.BlockSpec((RB, DN), lambda i: (i, 0)),
            pl.BlockSpec((DN, BP), lambda i: (0, 0)),
        ],
        out_specs=pl.BlockSpec((RB, 128), lambda i: (i, 0)),
        out_shape=jax.ShapeDtypeStruct((BP, 128), jnp.int32),
        scratch_shapes=[pltpu.VMEM((RB, BP), jnp.float32)],
    )(xp, xt)


def _k3a(xr, idxf, nc, ns):
    nw = nc * ns
    per_w = EP // nw
    ch = 256
    nch = per_w // ch
    mesh = plsc.VectorSubcoreMesh(core_axis_name="c", subcore_axis_name="s")

    @functools.partial(
        pl.kernel, mesh=mesh,
        out_type=jax.ShapeDtypeStruct((EP, DN), jnp.float32),
        scratch_types=[
            pltpu.VMEM((ch,), jnp.int32),
            pltpu.VMEM((ch, DN), jnp.float32),
            pltpu.SemaphoreType.DMA,
        ],
    )
    def gather_k(xr_hbm, idx_hbm, out_hbm, idx_v, rows_v, sem):
        wid = lax.axis_index("s") * nc + lax.axis_index("c")

        def body(j, carry):
            base = wid * per_w + j * ch
            pltpu.sync_copy(idx_hbm.at[pl.ds(base, ch)], idx_v)
            pltpu.async_copy(xr_hbm.at[idx_v], rows_v, sem).wait()
            pltpu.sync_copy(rows_v, out_hbm.at[pl.ds(base, ch)])
            return carry

        lax.fori_loop(0, nch, body, 0)

    return gather_k(xr, idxf)


def _k3b_body(g_ref, xl_ref, idx_ref, att_ref, gs_ref, val_ref, w_ref):
    i = pl.program_id(0)
    xl = xl_ref[...]
    attc = att_ref[...].reshape(DN, 1)
    gshift = gs_ref[0, 0]
    srci = lax.broadcasted_iota(jnp.int32, (RB, 1), 0) + i * RB
    for k in range(KN):
        xrg = g_ref[k]
        dsti = idx_ref[:, k:k + 1]
        z = xl + xrg
        lr = jnp.where(z > 0, z, NEG * z)
        lg = jnp.dot(lr, attc, preferred_element_type=jnp.float32)
        valid = jnp.logical_and(dsti != srci, srci < BN)
        w = jnp.where(valid, jnp.exp(lg - gshift), jnp.float32(0.0))
        val_ref[k] = w * xl
        w_ref[:, k:k + 1] = w


def _k3b(g3, xl, idxo, att2, gs):
    return pl.pallas_call(
        _k3b_body,
        grid=(NBLK,),
        in_specs=[
            pl.BlockSpec((KN, RB, DN), lambda i: (0, i, 0)),
            pl.BlockSpec((RB, DN), lambda i: (i, 0)),
            pl.BlockSpec((RB, 128), lambda i: (i, 0)),
            pl.BlockSpec((1, DN), lambda i: (0, 0)),
            pl.BlockSpec((8, DN), lambda i: (0, 0)),
        ],
        out_specs=(
            pl.BlockSpec((KN, RB, DN), lambda i: (0, i, 0)),
            pl.BlockSpec((RB, 128), lambda i: (i, 0)),
        ),
        out_shape=(
            jax.ShapeDtypeStruct((KN, BP, DN), jnp.float32),
            jax.ShapeDtypeStruct((BP, 128), jnp.float32),
        ),
    )(g3, xl, idxo, att2, gs)


def _k3c(val, wflat, idxf, zn, zd, nc, ns):
    per_w = EP // (nc * ns)
    ch = 256
    nch = per_w // ch
    half = EP // nc
    stripe = BP // ns
    mesh = plsc.VectorSubcoreMesh(core_axis_name="c", subcore_axis_name="s")

    @functools.partial(
        pl.kernel, mesh=mesh,
        out_type=(
            jax.ShapeDtypeStruct((2, BP, DN), jnp.float32),
            jax.ShapeDtypeStruct((2, BP), jnp.float32),
        ),
        scratch_types=[
            pltpu.VMEM((ch,), jnp.int32),
            pltpu.VMEM((ch, DN), jnp.float32),
            pltpu.VMEM((ch,), jnp.float32),
            pltpu.VMEM_SHARED((BP, DN), jnp.float32),
            pltpu.VMEM_SHARED((BP,), jnp.float32),
        ],
    )
    def scatter_k(val_hbm, w_hbm, idx_hbm, zn_hbm, zd_hbm, outn_hbm, outd_hbm,
                  idx_v, val_v, w_v, accn_sh, accd_sh):
        c = lax.axis_index("c")
        s = lax.axis_index("s")
        pltpu.sync_copy(zn_hbm.at[pl.ds(s * stripe, stripe)],
                        accn_sh.at[pl.ds(s * stripe, stripe)])
        pltpu.sync_copy(zd_hbm.at[pl.ds(s * stripe, stripe)],
                        accd_sh.at[pl.ds(s * stripe, stripe)])
        plsc.subcore_barrier()

        def body(j, carry):
            base = c * half + s * per_w + j * ch
            pltpu.sync_copy(idx_hbm.at[pl.ds(base, ch)], idx_v)
            pltpu.sync_copy(val_hbm.at[pl.ds(base, ch)], val_v)
            pltpu.sync_copy(w_hbm.at[pl.ds(base, ch)], w_v)
            pltpu.sync_copy(val_v, accn_sh.at[idx_v], add=True)
            pltpu.sync_copy(w_v, accd_sh.at[idx_v], add=True)
            return carry

        lax.fori_loop(0, nch, body, 0)
        plsc.subcore_barrier()
        pltpu.sync_copy(accn_sh.at[pl.ds(s * stripe, stripe)],
                        outn_hbm.at[c, pl.ds(s * stripe, stripe)])
        pltpu.sync_copy(accd_sh.at[pl.ds(s * stripe, stripe)],
                        outd_hbm.at[c, pl.ds(s * stripe, stripe)])

    return scatter_k(val, wflat, idxf, zn, zd)


def _k4_body(acc_ref, den_ref, self2_ref, bias_ref, out_ref):
    num = acc_ref[0] + acc_ref[1] + self2_ref[:, 0:DN]
    den = den_ref[...] + self2_ref[:, DN:2 * DN]
    out_ref[...] = num / (den + 1e-16) + bias_ref[...]


def _k4(acc, den_b, self2, bias2):
    blk = 1024
    return pl.pallas_call(
        _k4_body,
        grid=(BP // blk,),
        in_specs=[
            pl.BlockSpec((2, blk, DN), lambda i: (0, i, 0)),
            pl.BlockSpec((blk, DN), lambda i: (i, 0)),
            pl.BlockSpec((blk, 2 * DN), lambda i: (i, 0)),
            pl.BlockSpec((1, DN), lambda i: (0, 0)),
        ],
        out_specs=pl.BlockSpec((blk, DN), lambda i: (i, 0)),
        out_shape=jax.ShapeDtypeStruct((BP, DN), jnp.float32),
    )(acc, den_b, self2, bias2)


def kernel(x, Wl, bl, Wr, br, att, bias):
    info = plsc.get_sparse_core_info()
    nc, ns = info.num_cores, info.num_subcores

    xp = jnp.pad(x, ((0, BP - BN), (0, 0)))
    xl, xr, gs, self2 = _k1(xp, Wl.T, bl.reshape(1, DN), Wr.T,
                            br.reshape(1, DN), att.reshape(1, DN))
    idxo = _k2(xp, xp.T)
    idxf = jnp.transpose(idxo[:, :KN]).reshape(-1)
    g = _k3a(xr, idxf, nc, ns)
    val3, wmat = _k3b(g.reshape(KN, BP, DN), xl, idxo, att.reshape(1, DN), gs)
    wflat = jnp.transpose(wmat[:, :KN]).reshape(-1)
    zn = jnp.zeros((BP, DN), jnp.float32)
    zd = jnp.zeros((BP,), jnp.float32)
    acc, accd = _k3c(val3.reshape(EP, DN), wflat, idxf, zn, zd, nc, ns)
    den_b = jnp.broadcast_to((accd[0] + accd[1]).reshape(BP, 1), (BP, DN))
    out = _k4(acc, den_b, self2, bias.reshape(1, DN))
    return out[:BN]


# speedup vs baseline: 7.3905x; 1.9775x over previous
import functools
import math

import jax
import jax.numpy as jnp
from jax import lax
from jax.experimental import pallas as pl
from jax.experimental.pallas import tpu as pltpu
from jax.experimental.pallas import tpu_sc as plsc

BN = 10000
DN = 128
KN = 16
BP = 10240
RB = 256
NBLK = BP // RB
EP = KN * BP
NEG = 0.2
NEG_INF = float("-inf")


def _k1_body(xp_ref, wlt_ref, bl_ref, wrt_ref, br_ref, att_ref,
             xl_ref, xr_ref, gs_ref, self2_ref):
    xp = xp_ref[...]
    xl = jnp.dot(xp, wlt_ref[...], preferred_element_type=jnp.float32) + bl_ref[...]
    xr = jnp.dot(xp, wrt_ref[...], preferred_element_type=jnp.float32) + br_ref[...]
    attc = att_ref[...].reshape(DN, 1)
    aabs = jnp.abs(attc)
    al = jnp.dot(xl, attc, preferred_element_type=jnp.float32)
    p = jnp.dot(jnp.abs(xl), aabs, preferred_element_type=jnp.float32)
    ar = jnp.dot(xr, attc, preferred_element_type=jnp.float32)
    q = jnp.dot(jnp.abs(xr), aabs, preferred_element_type=jnp.float32)
    rowid = lax.broadcasted_iota(jnp.int32, (BP, 1), 0)
    ok = rowid < BN
    gsrc = jnp.max(jnp.where(ok, 0.6 * al + 0.4 * p, jnp.float32(-1e30)))
    gdst = jnp.max(jnp.where(ok, 0.6 * ar + 0.4 * q, jnp.float32(-1e30)))
    gshift = gsrc + gdst
    z = xl + xr
    lr = jnp.where(z > 0, z, NEG * z)
    ls = jnp.dot(lr, attc, preferred_element_type=jnp.float32)
    selfw = jnp.exp(ls - gshift)
    xl_ref[...] = xl
    xr_ref[...] = xr
    gs_ref[...] = jnp.full((8, DN), gshift, jnp.float32)
    self2_ref[...] = jnp.concatenate(
        [selfw * xl, jnp.broadcast_to(selfw, (BP, DN))], axis=1)


def _k1(xp, wlt, bl2, wrt, br2, att2):
    return pl.pallas_call(
        _k1_body,
        out_shape=(
            jax.ShapeDtypeStruct((BP, DN), jnp.float32),
            jax.ShapeDtypeStruct((BP, DN), jnp.float32),
            jax.ShapeDtypeStruct((8, DN), jnp.float32),
            jax.ShapeDtypeStruct((BP, 2 * DN), jnp.float32),
        ),
    )(xp, wlt, bl2, wrt, br2, att2)


NSG = BP // 128
NR = 4


def _k2_body(xrow_ref, xt_ref, idx_ref, s_ref):
    s_ref[...] = jnp.dot(xrow_ref[...], xt_ref[...],
                         preferred_element_type=jnp.float32)
    big = jnp.int32(1 << 30)
    gid3 = (lax.broadcasted_iota(jnp.int32, (RB, NSG, 128), 1) * 128
            + lax.broadcasted_iota(jnp.int32, (RB, NSG, 128), 2))
    s3 = jnp.where(gid3 < BN, s_ref[...].reshape(RB, NSG, 128), NEG_INF)

    cvs, cis = [], []
    for j in range(NR):
        cm = jnp.max(s3, axis=1, keepdims=True)
        ga = jnp.min(jnp.where(s3 == cm, gid3, big), axis=1, keepdims=True)
        cvs.append(cm.reshape(RB, 128))
        cis.append(ga.reshape(RB, 128))
        s3 = jnp.where(gid3 == ga, NEG_INF, s3)
    cv = jnp.concatenate(cvs, axis=1)
    ci = jnp.concatenate(cis, axis=1)

    m = None
    for k in range(KN):
        m = jnp.max(cv, axis=1, keepdims=True)
        gi = jnp.min(jnp.where(cv == m, ci, big), axis=1, keepdims=True)
        idx_ref[:, k:k + 1] = gi
        cv = jnp.where(ci == gi, NEG_INF, cv)

    rmax = jnp.max(jnp.max(s3, axis=1, keepdims=True), axis=2, keepdims=True)
    bad = jnp.max((rmax.reshape(RB, 1) >= m).astype(jnp.int32)) > 0

    @pl.when(bad)
    def _fallback():
        colid = lax.broadcasted_iota(jnp.int32, (RB, BP), 1)
        s = jnp.where(colid < BN, s_ref[...], NEG_INF)
        for k in range(KN):
            a = jnp.argmax(s, axis=1).astype(jnp.int32)[:, None]
            idx_ref[:, k:k + 1] = a
            s = jnp.where(colid == a, NEG_INF, s)


def _k2(xp, xt):
    return pl.pallas_call(
        _k2_body,
        grid=(NBLK,),
        in_specs=[
            pl.BlockSpec((RB, DN), lambda i: (i, 0)),
            pl.BlockSpec((DN, BP), lambda i: (0, 0)),
        ],
        out_specs=pl.BlockSpec((RB, 128), lambda i: (i, 0)),
        out_shape=jax.ShapeDtypeStruct((BP, 128), jnp.int32),
        scratch_shapes=[pltpu.VMEM((RB, BP), jnp.float32)],
    )(xp, xt)


def _k3a(xr, idxf, nc, ns):
    nw = nc * ns
    per_w = EP // nw
    ch = 256
    nch = per_w // ch
    mesh = plsc.VectorSubcoreMesh(core_axis_name="c", subcore_axis_name="s")

    @functools.partial(
        pl.kernel, mesh=mesh,
        out_type=jax.ShapeDtypeStruct((EP, DN), jnp.float32),
        scratch_types=[
            pltpu.VMEM((ch,), jnp.int32),
            pltpu.VMEM((ch, DN), jnp.float32),
            pltpu.SemaphoreType.DMA,
        ],
    )
    def gather_k(xr_hbm, idx_hbm, out_hbm, idx_v, rows_v, sem):
        wid = lax.axis_index("s") * nc + lax.axis_index("c")

        def body(j, carry):
            base = wid * per_w + j * ch
            pltpu.sync_copy(idx_hbm.at[pl.ds(base, ch)], idx_v)
            pltpu.async_copy(xr_hbm.at[idx_v], rows_v, sem).wait()
            pltpu.sync_copy(rows_v, out_hbm.at[pl.ds(base, ch)])
            return carry

        lax.fori_loop(0, nch, body, 0)

    return gather_k(xr, idxf)


def _k3b_body(g_ref, xl_ref, idx_ref, att_ref, gs_ref, val_ref, w_ref):
    i = pl.program_id(0)
    xl = xl_ref[...]
    attc = att_ref[...].reshape(DN, 1)
    gshift = gs_ref[0, 0]
    srci = lax.broadcasted_iota(jnp.int32, (RB, 1), 0) + i * RB
    for k in range(KN):
        xrg = g_ref[k]
        dsti = idx_ref[:, k:k + 1]
        z = xl + xrg
        lr = jnp.where(z > 0, z, NEG * z)
        lg = jnp.dot(lr, attc, preferred_element_type=jnp.float32)
        valid = jnp.logical_and(dsti != srci, srci < BN)
        w = jnp.where(valid, jnp.exp(lg - gshift), jnp.float32(0.0))
        val_ref[k] = w * xl
        w_ref[:, k:k + 1] = w


def _k3b(g3, xl, idxo, att2, gs):
    return pl.pallas_call(
        _k3b_body,
        grid=(NBLK,),
        in_specs=[
            pl.BlockSpec((KN, RB, DN), lambda i: (0, i, 0)),
            pl.BlockSpec((RB, DN), lambda i: (i, 0)),
            pl.BlockSpec((RB, 128), lambda i: (i, 0)),
            pl.BlockSpec((1, DN), lambda i: (0, 0)),
            pl.BlockSpec((8, DN), lambda i: (0, 0)),
        ],
        out_specs=(
            pl.BlockSpec((KN, RB, DN), lambda i: (0, i, 0)),
            pl.BlockSpec((RB, 128), lambda i: (i, 0)),
        ),
        out_shape=(
            jax.ShapeDtypeStruct((KN, BP, DN), jnp.float32),
            jax.ShapeDtypeStruct((BP, 128), jnp.float32),
        ),
    )(g3, xl, idxo, att2, gs)


def _k3c(val, wflat, idxf, zn, zd, nc, ns):
    per_w = EP // (nc * ns)
    ch = 256
    nch = per_w // ch
    half = EP // nc
    stripe = BP // ns
    mesh = plsc.VectorSubcoreMesh(core_axis_name="c", subcore_axis_name="s")

    @functools.partial(
        pl.kernel, mesh=mesh,
        out_type=(
            jax.ShapeDtypeStruct((2, BP, DN), jnp.float32),
            jax.ShapeDtypeStruct((2, BP), jnp.float32),
        ),
        scratch_types=[
            pltpu.VMEM((ch,), jnp.int32),
            pltpu.VMEM((ch, DN), jnp.float32),
            pltpu.VMEM((ch,), jnp.float32),
            pltpu.VMEM_SHARED((BP, DN), jnp.float32),
            pltpu.VMEM_SHARED((BP,), jnp.float32),
        ],
    )
    def scatter_k(val_hbm, w_hbm, idx_hbm, zn_hbm, zd_hbm, outn_hbm, outd_hbm,
                  idx_v, val_v, w_v, accn_sh, accd_sh):
        c = lax.axis_index("c")
        s = lax.axis_index("s")
        pltpu.sync_copy(zn_hbm.at[pl.ds(s * stripe, stripe)],
                        accn_sh.at[pl.ds(s * stripe, stripe)])
        pltpu.sync_copy(zd_hbm.at[pl.ds(s * stripe, stripe)],
                        accd_sh.at[pl.ds(s * stripe, stripe)])
        plsc.subcore_barrier()

        def body(j, carry):
            base = c * half + s * per_w + j * ch
            pltpu.sync_copy(idx_hbm.at[pl.ds(base, ch)], idx_v)
            pltpu.sync_copy(val_hbm.at[pl.ds(base, ch)], val_v)
            pltpu.sync_copy(w_hbm.at[pl.ds(base, ch)], w_v)
            pltpu.sync_copy(val_v, accn_sh.at[idx_v], add=True)
            pltpu.sync_copy(w_v, accd_sh.at[idx_v], add=True)
            return carry

        lax.fori_loop(0, nch, body, 0)
        plsc.subcore_barrier()
        pltpu.sync_copy(accn_sh.at[pl.ds(s * stripe, stripe)],
                        outn_hbm.at[c, pl.ds(s * stripe, stripe)])
        pltpu.sync_copy(accd_sh.at[pl.ds(s * stripe, stripe)],
                        outd_hbm.at[c, pl.ds(s * stripe, stripe)])

    return scatter_k(val, wflat, idxf, zn, zd)


def _k4_body(acc_ref, den_ref, self2_ref, bias_ref, out_ref):
    num = acc_ref[0] + acc_ref[1] + self2_ref[:, 0:DN]
    den = den_ref[...] + self2_ref[:, DN:2 * DN]
    out_ref[...] = num / (den + 1e-16) + bias_ref[...]


def _k4(acc, den_b, self2, bias2):
    blk = 1024
    return pl.pallas_call(
        _k4_body,
        grid=(BP // blk,),
        in_specs=[
            pl.BlockSpec((2, blk, DN), lambda i: (0, i, 0)),
            pl.BlockSpec((blk, DN), lambda i: (i, 0)),
            pl.BlockSpec((blk, 2 * DN), lambda i: (i, 0)),
            pl.BlockSpec((1, DN), lambda i: (0, 0)),
        ],
        out_specs=pl.BlockSpec((blk, DN), lambda i: (i, 0)),
        out_shape=jax.ShapeDtypeStruct((BP, DN), jnp.float32),
    )(acc, den_b, self2, bias2)


def kernel(x, Wl, bl, Wr, br, att, bias):
    info = plsc.get_sparse_core_info()
    nc, ns = info.num_cores, info.num_subcores

    xp = jnp.pad(x, ((0, BP - BN), (0, 0)))
    xl, xr, gs, self2 = _k1(xp, Wl.T, bl.reshape(1, DN), Wr.T,
                            br.reshape(1, DN), att.reshape(1, DN))
    idxo = _k2(xp, xp.T)
    idxf = jnp.transpose(idxo[:, :KN]).reshape(-1)
    g = _k3a(xr, idxf, nc, ns)
    val3, wmat = _k3b(g.reshape(KN, BP, DN), xl, idxo, att.reshape(1, DN), gs)
    wflat = jnp.transpose(wmat[:, :KN]).reshape(-1)
    zn = jnp.zeros((BP, DN), jnp.float32)
    zd = jnp.zeros((BP,), jnp.float32)
    acc, accd = _k3c(val3.reshape(EP, DN), wflat, idxf, zn, zd, nc, ns)
    den_b = jnp.broadcast_to((accd[0] + accd[1]).reshape(BP, 1), (BP, DN))
    out = _k4(acc, den_b, self2, bias.reshape(1, DN))
    return out[:BN]


# speedup vs baseline: 7.3975x; 1.0009x over previous
import functools
import math

import jax
import jax.numpy as jnp
from jax import lax
from jax.experimental import pallas as pl
from jax.experimental.pallas import tpu as pltpu
from jax.experimental.pallas import tpu_sc as plsc

BN = 10000
DN = 128
KN = 16
BP = 10240
RB = 256
NBLK = BP // RB
EP = KN * BP
NEG = 0.2
NEG_INF = float("-inf")


def _k1_body(xp_ref, wlt_ref, bl_ref, wrt_ref, br_ref, att_ref,
             xl_ref, xr_ref, gs_ref, self2_ref):
    xp = xp_ref[...]
    xl = jnp.dot(xp, wlt_ref[...], preferred_element_type=jnp.float32) + bl_ref[...]
    xr = jnp.dot(xp, wrt_ref[...], preferred_element_type=jnp.float32) + br_ref[...]
    attc = att_ref[...].reshape(DN, 1)
    aabs = jnp.abs(attc)
    al = jnp.dot(xl, attc, preferred_element_type=jnp.float32)
    p = jnp.dot(jnp.abs(xl), aabs, preferred_element_type=jnp.float32)
    ar = jnp.dot(xr, attc, preferred_element_type=jnp.float32)
    q = jnp.dot(jnp.abs(xr), aabs, preferred_element_type=jnp.float32)
    rowid = lax.broadcasted_iota(jnp.int32, (BP, 1), 0)
    ok = rowid < BN
    gsrc = jnp.max(jnp.where(ok, 0.6 * al + 0.4 * p, jnp.float32(-1e30)))
    gdst = jnp.max(jnp.where(ok, 0.6 * ar + 0.4 * q, jnp.float32(-1e30)))
    gshift = gsrc + gdst
    z = xl + xr
    lr = jnp.where(z > 0, z, NEG * z)
    ls = jnp.dot(lr, attc, preferred_element_type=jnp.float32)
    selfw = jnp.exp(ls - gshift)
    xl_ref[...] = xl
    xr_ref[...] = xr
    gs_ref[...] = jnp.full((8, DN), gshift, jnp.float32)
    self2_ref[...] = jnp.concatenate(
        [selfw * xl, jnp.broadcast_to(selfw, (BP, DN))], axis=1)


def _k1(xp, wlt, bl2, wrt, br2, att2):
    return pl.pallas_call(
        _k1_body,
        out_shape=(
            jax.ShapeDtypeStruct((BP, DN), jnp.float32),
            jax.ShapeDtypeStruct((BP, DN), jnp.float32),
            jax.ShapeDtypeStruct((8, DN), jnp.float32),
            jax.ShapeDtypeStruct((BP, 2 * DN), jnp.float32),
        ),
    )(xp, wlt, bl2, wrt, br2, att2)


NSG = BP // 128
NR = 4


def _k2_body(xrow_ref, xt_ref, idx_ref, s_ref):
    s_ref[...] = jnp.dot(xrow_ref[...], xt_ref[...],
                         preferred_element_type=jnp.float32)
    big = jnp.int32(1 << 30)
    gid3 = (lax.broadcasted_iota(jnp.int32, (RB, NSG, 128), 1) * 128
            + lax.broadcasted_iota(jnp.int32, (RB, NSG, 128), 2))
    s3 = jnp.where(gid3 < BN, s_ref[...].reshape(RB, NSG, 128), NEG_INF)

    cvs, cis = [], []
    for j in range(NR):
        cm = jnp.max(s3, axis=1, keepdims=True)
        ga = jnp.min(jnp.where(s3 == cm, gid3, big), axis=1, keepdims=True)
        cvs.append(cm.reshape(RB, 128))
        cis.append(ga.reshape(RB, 128))
        s3 = jnp.where(gid3 == ga, NEG_INF, s3)
    cv = jnp.concatenate(cvs, axis=1)
    ci = jnp.concatenate(cis, axis=1)

    m = None
    for k in range(KN):
        m = jnp.max(cv, axis=1, keepdims=True)
        gi = jnp.min(jnp.where(cv == m, ci, big), axis=1, keepdims=True)
        idx_ref[:, k:k + 1] = gi
        cv = jnp.where(ci == gi, NEG_INF, cv)

    rmax = jnp.max(jnp.max(s3, axis=1, keepdims=True), axis=2, keepdims=True)
    bad = jnp.max((rmax.reshape(RB, 1) >= m).astype(jnp.int32)) > 0

    @pl.when(bad)
    def _fallback():
        colid = lax.broadcasted_iota(jnp.int32, (RB, BP), 1)
        s = jnp.where(colid < BN, s_ref[...], NEG_INF)
        for k in range(KN):
            a = jnp.argmax(s, axis=1).astype(jnp.int32)[:, None]
            idx_ref[:, k:k + 1] = a
            s = jnp.where(colid == a, NEG_INF, s)


def _k2(xp, xt):
    return pl.pallas_call(
        _k2_body,
        grid=(NBLK,),
        in_specs=[
            pl.BlockSpec((RB, DN), lambda i: (i, 0)),
            pl.BlockSpec((DN, BP), lambda i: (0, 0)),
        ],
        out_specs=pl.BlockSpec((RB, 128), lambda i: (i, 0)),
        out_shape=jax.ShapeDtypeStruct((BP, 128), jnp.int32),
        scratch_shapes=[pltpu.VMEM((RB, BP), jnp.float32)],
    )(xp, xt)


def _k3a(xr, idxf, nc, ns):
    nw = nc * ns
    per_w = EP // nw
    ch = 256
    nch = per_w // ch
    mesh = plsc.VectorSubcoreMesh(core_axis_name="c", subcore_axis_name="s")

    @functools.partial(
        pl.kernel, mesh=mesh,
        out_type=jax.ShapeDtypeStruct((EP, DN), jnp.float32),
        scratch_types=[
            pltpu.VMEM((ch,), jnp.int32),
            pltpu.VMEM((ch, DN), jnp.float32),
            pltpu.SemaphoreType.DMA,
        ],
    )
    def gather_k(xr_hbm, idx_hbm, out_hbm, idx_v, rows_v, sem):
        wid = lax.axis_index("s") * nc + lax.axis_index("c")

        def body(j, carry):
            base = wid * per_w + j * ch
            pltpu.sync_copy(idx_hbm.at[pl.ds(base, ch)], idx_v)
            pltpu.async_copy(xr_hbm.at[idx_v], rows_v, sem).wait()
            pltpu.sync_copy(rows_v, out_hbm.at[pl.ds(base, ch)])
            return carry

        lax.fori_loop(0, nch, body, 0)

    return gather_k(xr, idxf)


def _k3b_body(g_ref, xl_ref, idx_ref, att_ref, gs_ref, val_ref, w_ref):
    i = pl.program_id(0)
    xl = xl_ref[...]
    attc = att_ref[...].reshape(DN, 1)
    gshift = gs_ref[0, 0]
    srci = lax.broadcasted_iota(jnp.int32, (RB, 1), 0) + i * RB
    lrs = []
    for k in range(KN):
        z = xl + g_ref[k]
        lrs.append(jnp.where(z > 0, z, NEG * z))
    lg_all = jnp.dot(jnp.concatenate(lrs, axis=0), attc,
                     preferred_element_type=jnp.float32)
    for k in range(KN):
        dsti = idx_ref[:, k:k + 1]
        lg = lg_all[k * RB:(k + 1) * RB]
        valid = jnp.logical_and(dsti != srci, srci < BN)
        w = jnp.where(valid, jnp.exp(lg - gshift), jnp.float32(0.0))
        val_ref[k] = w * xl
        w_ref[:, k:k + 1] = w


def _k3b(g3, xl, idxo, att2, gs):
    return pl.pallas_call(
        _k3b_body,
        grid=(NBLK,),
        in_specs=[
            pl.BlockSpec((KN, RB, DN), lambda i: (0, i, 0)),
            pl.BlockSpec((RB, DN), lambda i: (i, 0)),
            pl.BlockSpec((RB, 128), lambda i: (i, 0)),
            pl.BlockSpec((1, DN), lambda i: (0, 0)),
            pl.BlockSpec((8, DN), lambda i: (0, 0)),
        ],
        out_specs=(
            pl.BlockSpec((KN, RB, DN), lambda i: (0, i, 0)),
            pl.BlockSpec((RB, 128), lambda i: (i, 0)),
        ),
        out_shape=(
            jax.ShapeDtypeStruct((KN, BP, DN), jnp.float32),
            jax.ShapeDtypeStruct((BP, 128), jnp.float32),
        ),
    )(g3, xl, idxo, att2, gs)


def _k3c(val, wflat, idxf, zn, zd, nc, ns):
    per_w = EP // (nc * ns)
    ch = 256
    nch = per_w // ch
    half = EP // nc
    stripe = BP // ns
    mesh = plsc.VectorSubcoreMesh(core_axis_name="c", subcore_axis_name="s")

    @functools.partial(
        pl.kernel, mesh=mesh,
        out_type=(
            jax.ShapeDtypeStruct((2, BP, DN), jnp.float32),
            jax.ShapeDtypeStruct((2, BP), jnp.float32),
        ),
        scratch_types=[
            pltpu.VMEM((ch,), jnp.int32),
            pltpu.VMEM((ch, DN), jnp.float32),
            pltpu.VMEM((ch,), jnp.float32),
            pltpu.VMEM_SHARED((BP, DN), jnp.float32),
            pltpu.VMEM_SHARED((BP,), jnp.float32),
        ],
    )
    def scatter_k(val_hbm, w_hbm, idx_hbm, zn_hbm, zd_hbm, outn_hbm, outd_hbm,
                  idx_v, val_v, w_v, accn_sh, accd_sh):
        c = lax.axis_index("c")
        s = lax.axis_index("s")
        pltpu.sync_copy(zn_hbm.at[pl.ds(s * stripe, stripe)],
                        accn_sh.at[pl.ds(s * stripe, stripe)])
        pltpu.sync_copy(zd_hbm.at[pl.ds(s * stripe, stripe)],
                        accd_sh.at[pl.ds(s * stripe, stripe)])
        plsc.subcore_barrier()

        def body(j, carry):
            base = c * half + s * per_w + j * ch
            pltpu.sync_copy(idx_hbm.at[pl.ds(base, ch)], idx_v)
            pltpu.sync_copy(val_hbm.at[pl.ds(base, ch)], val_v)
            pltpu.sync_copy(w_hbm.at[pl.ds(base, ch)], w_v)
            pltpu.sync_copy(val_v, accn_sh.at[idx_v], add=True)
            pltpu.sync_copy(w_v, accd_sh.at[idx_v], add=True)
            return carry

        lax.fori_loop(0, nch, body, 0)
        plsc.subcore_barrier()
        pltpu.sync_copy(accn_sh.at[pl.ds(s * stripe, stripe)],
                        outn_hbm.at[c, pl.ds(s * stripe, stripe)])
        pltpu.sync_copy(accd_sh.at[pl.ds(s * stripe, stripe)],
                        outd_hbm.at[c, pl.ds(s * stripe, stripe)])

    return scatter_k(val, wflat, idxf, zn, zd)


def _k4_body(acc_ref, den_ref, self2_ref, bias_ref, out_ref):
    num = acc_ref[0] + acc_ref[1] + self2_ref[:, 0:DN]
    den = den_ref[...] + self2_ref[:, DN:2 * DN]
    out_ref[...] = num / (den + 1e-16) + bias_ref[...]


def _k4(acc, den_b, self2, bias2):
    blk = 1024
    return pl.pallas_call(
        _k4_body,
        grid=(BP // blk,),
        in_specs=[
            pl.BlockSpec((2, blk, DN), lambda i: (0, i, 0)),
            pl.BlockSpec((blk, DN), lambda i: (i, 0)),
            pl.BlockSpec((blk, 2 * DN), lambda i: (i, 0)),
            pl.BlockSpec((1, DN), lambda i: (0, 0)),
        ],
        out_specs=pl.BlockSpec((blk, DN), lambda i: (i, 0)),
        out_shape=jax.ShapeDtypeStruct((BP, DN), jnp.float32),
    )(acc, den_b, self2, bias2)


def kernel(x, Wl, bl, Wr, br, att, bias):
    info = plsc.get_sparse_core_info()
    nc, ns = info.num_cores, info.num_subcores

    xp = jnp.pad(x, ((0, BP - BN), (0, 0)))
    xl, xr, gs, self2 = _k1(xp, Wl.T, bl.reshape(1, DN), Wr.T,
                            br.reshape(1, DN), att.reshape(1, DN))
    idxo = _k2(xp, xp.T)
    idxf = jnp.transpose(idxo[:, :KN]).reshape(-1)
    g = _k3a(xr, idxf, nc, ns)
    val3, wmat = _k3b(g.reshape(KN, BP, DN), xl, idxo, att.reshape(1, DN), gs)
    wflat = jnp.transpose(wmat[:, :KN]).reshape(-1)
    zn = jnp.zeros((BP, DN), jnp.float32)
    zd = jnp.zeros((BP,), jnp.float32)
    acc, accd = _k3c(val3.reshape(EP, DN), wflat, idxf, zn, zd, nc, ns)
    den_b = jnp.broadcast_to((accd[0] + accd[1]).reshape(BP, 1), (BP, DN))
    out = _k4(acc, den_b, self2, bias.reshape(1, DN))
    return out[:BN]
